# Initial kernel scaffold; baseline (speedup 1.0000x reference)
import jax
import jax.numpy as jnp
from jax.experimental import pallas as pl


def kernel(x, codebooks, dec_w, dec_b, w1, b1, w2, b2):
    raise NotImplementedError("write your pallas kernel here")


# speedup vs baseline: 450.1662x; 450.1662x over previous
import functools

import jax
import jax.numpy as jnp
import numpy as np
from jax import lax
from jax.experimental import pallas as pl
from jax.experimental.pallas import tpu as pltpu
from jax.experimental.pallas import tpu_sc as plsc

_NUM_LODS = 16
_TABLE = 4096
_N = 262144
_HIDDEN = 16
_OUT = 3
_MIN_RES, _MAX_RES = 16, 512

_bf = np.exp((np.log(_MAX_RES) - np.log(_MIN_RES)) / (_NUM_LODS - 1))
_RES = [int(np.floor(_MIN_RES * (_bf ** l))) for l in range(_NUM_LODS)]

_NC, _NS, _L = 2, 16, 16
_NW = _NC * _NS
_PPW = _N // _NW
_BLK = 2048
_CH = _BLK // _L

_mesh = plsc.VectorSubcoreMesh(core_axis_name="c", subcore_axis_name="s")


@functools.partial(
    pl.kernel,
    mesh=_mesh,
    compiler_params=pltpu.CompilerParams(needs_layout_passes=False),
    out_type=jax.ShapeDtypeStruct((_N * _NUM_LODS,), jnp.float32),
    scratch_types=[
        pltpu.VMEM((_NUM_LODS * _TABLE,), jnp.float32),
        pltpu.VMEM((_BLK,), jnp.float32),
        pltpu.VMEM((_BLK,), jnp.float32),
        pltpu.VMEM((_BLK * _NUM_LODS,), jnp.float32),
    ],
)
def _sc_latents(x0_hbm, x1_hbm, tab_hbm, out_hbm, tab_v, x0_v, x1_v, fe_v):
    wid = lax.axis_index("s") * _NC + lax.axis_index("c")
    pltpu.sync_copy(tab_hbm, tab_v)
    lane = lax.iota(jnp.int32, _L)
    kmul = jnp.uint32(2654435761)
    mask = jnp.uint32(_TABLE - 1)
    one = jnp.uint32(1)

    for blk in range(_PPW // _BLK):
        off = wid * _PPW + blk * _BLK
        pltpu.sync_copy(x0_hbm.at[pl.ds(off, _BLK)], x0_v)
        pltpu.sync_copy(x1_hbm.at[pl.ds(off, _BLK)], x1_v)

        def body(i, carry):
            xv = x0_v[pl.ds(i * _L, _L)]
            yv = x1_v[pl.ds(i * _L, _L)]
            base = i * (_L * _NUM_LODS) + lane * _NUM_LODS
            for l in range(_NUM_LODS):
                r = float(_RES[l])
                px = xv * r
                py = yv * r
                ix = px.astype(jnp.int32)
                iy = py.astype(jnp.int32)
                fx = px - ix.astype(jnp.float32)
                fy = py - iy.astype(jnp.float32)
                ux = ix.astype(jnp.uint32)
                uy = iy.astype(jnp.uint32)
                hy0 = uy * kmul
                hy1 = (uy + one) * kmul
                ux1 = ux + one
                off_l = jnp.int32(l * _TABLE)
                i00 = ((ux ^ hy0) & mask).astype(jnp.int32) + off_l
                i10 = ((ux1 ^ hy0) & mask).astype(jnp.int32) + off_l
                i01 = ((ux ^ hy1) & mask).astype(jnp.int32) + off_l
                i11 = ((ux1 ^ hy1) & mask).astype(jnp.int32) + off_l
                f00 = plsc.load_gather(tab_v, [i00])
                f10 = plsc.load_gather(tab_v, [i10])
                f01 = plsc.load_gather(tab_v, [i01])
                f11 = plsc.load_gather(tab_v, [i11])
                a = f00 + fx * (f10 - f00)
                b = f01 + fx * (f11 - f01)
                lat = a + fy * (b - a)
                plsc.store_scatter(fe_v, [base + l], lat)
            return carry

        lax.fori_loop(0, _CH, body, 0)
        pltpu.sync_copy(
            fe_v, out_hbm.at[pl.ds(off * _NUM_LODS, _BLK * _NUM_LODS)])


def _mlp_body(f_ref, w1_ref, b1_ref, w2_ref, b2_ref, o_ref):
    f = f_ref[...]
    h = jnp.dot(f, w1_ref[...], preferred_element_type=jnp.float32)
    h = jnp.maximum(h + b1_ref[...], 0.0)
    g = jnp.dot(h, w2_ref[...], preferred_element_type=jnp.float32)
    g = g + b2_ref[...]
    o_ref[...] = 1.0 / (1.0 + jnp.exp(-g))


_PACK = 16
_ROWS = _N // _PACK
_LANES = _PACK * _NUM_LODS
_OLANES = _PACK * _OUT


def kernel(x, codebooks, dec_w, dec_b, w1, b1, w2, b2):
    x0 = x[:, 0]
    x1 = x[:, 1]
    tab = codebooks.reshape(_NUM_LODS * _TABLE)
    w1f = w1 * dec_w[:, None]
    b1f = b1 + dec_b @ w1
    eye = jnp.eye(_PACK, dtype=jnp.float32)
    w1_bd = jnp.kron(eye, w1f)
    b1_bd = jnp.tile(b1f, _PACK).reshape(1, _LANES)
    w2_bd = jnp.kron(eye, w2)
    b2_bd = jnp.tile(b2, _PACK).reshape(1, _OLANES)

    lat = _sc_latents(x0, x1, tab).reshape(_ROWS, _LANES)

    bm = 2048
    out = pl.pallas_call(
        _mlp_body,
        grid=(_ROWS // bm,),
        in_specs=[
            pl.BlockSpec((bm, _LANES), lambda i: (i, 0)),
            pl.BlockSpec((_LANES, _LANES), lambda i: (0, 0)),
            pl.BlockSpec((1, _LANES), lambda i: (0, 0)),
            pl.BlockSpec((_LANES, _OLANES), lambda i: (0, 0)),
            pl.BlockSpec((1, _OLANES), lambda i: (0, 0)),
        ],
        out_specs=pl.BlockSpec((bm, _OLANES), lambda i: (i, 0)),
        out_shape=jax.ShapeDtypeStruct((_ROWS, _OLANES), jnp.float32),
    )(lat, w1_bd, b1_bd, w2_bd, b2_bd)
    return out.reshape(_N, _OUT)


# speedup vs baseline: 898.8559x; 1.9967x over previous
import functools

import jax
import jax.numpy as jnp
import numpy as np
from jax import lax
from jax.experimental import pallas as pl
from jax.experimental.pallas import tpu as pltpu
from jax.experimental.pallas import tpu_sc as plsc

_NUM_LODS = 16
_TABLE = 4096
_N = 262144
_HIDDEN = 16
_OUT = 3
_MIN_RES, _MAX_RES = 16, 512

_bf = np.exp((np.log(_MAX_RES) - np.log(_MIN_RES)) / (_NUM_LODS - 1))
_RES = [int(np.floor(_MIN_RES * (_bf ** l))) for l in range(_NUM_LODS)]

_NC, _NS, _L = 2, 16, 16
_NW = _NC * _NS
_PPW = _N // _NW
_BLK = 2048
_CH = _BLK // _L

_mesh = plsc.VectorSubcoreMesh(core_axis_name="c", subcore_axis_name="s")


@functools.partial(
    pl.kernel,
    mesh=_mesh,
    compiler_params=pltpu.CompilerParams(needs_layout_passes=False),
    out_type=jax.ShapeDtypeStruct((_NUM_LODS, _N), jnp.float32),
    scratch_types=[
        pltpu.VMEM((_NUM_LODS * _TABLE,), jnp.float32),
        pltpu.VMEM((_BLK,), jnp.float32),
        pltpu.VMEM((_BLK,), jnp.float32),
        pltpu.VMEM((_NUM_LODS, _BLK), jnp.float32),
    ],
)
def _sc_latents(x0_hbm, x1_hbm, tab_hbm, out_hbm, tab_v, x0_v, x1_v, fe_v):
    wid = lax.axis_index("s") * _NC + lax.axis_index("c")
    pltpu.sync_copy(tab_hbm, tab_v)
    kmul = jnp.uint32(2654435761)
    mask = jnp.uint32(_TABLE - 1)
    one = jnp.uint32(1)

    for blk in range(_PPW // _BLK):
        off = wid * _PPW + blk * _BLK
        pltpu.sync_copy(x0_hbm.at[pl.ds(off, _BLK)], x0_v)
        pltpu.sync_copy(x1_hbm.at[pl.ds(off, _BLK)], x1_v)

        def body(i, carry):
            xv = x0_v[pl.ds(i * _L, _L)]
            yv = x1_v[pl.ds(i * _L, _L)]
            for l in range(_NUM_LODS):
                r = float(_RES[l])
                px = xv * r
                py = yv * r
                ix = px.astype(jnp.int32)
                iy = py.astype(jnp.int32)
                fx = px - ix.astype(jnp.float32)
                fy = py - iy.astype(jnp.float32)
                ux = ix.astype(jnp.uint32)
                uy = iy.astype(jnp.uint32)
                hy0 = uy * kmul
                hy1 = (uy + one) * kmul
                ux1 = ux + one
                off_l = jnp.int32(l * _TABLE)
                i00 = ((ux ^ hy0) & mask).astype(jnp.int32) + off_l
                i10 = ((ux1 ^ hy0) & mask).astype(jnp.int32) + off_l
                i01 = ((ux ^ hy1) & mask).astype(jnp.int32) + off_l
                i11 = ((ux1 ^ hy1) & mask).astype(jnp.int32) + off_l
                f00 = plsc.load_gather(tab_v, [i00])
                f10 = plsc.load_gather(tab_v, [i10])
                f01 = plsc.load_gather(tab_v, [i01])
                f11 = plsc.load_gather(tab_v, [i11])
                a = f00 + fx * (f10 - f00)
                b = f01 + fx * (f11 - f01)
                fe_v[l, pl.ds(i * _L, _L)] = a + fy * (b - a)
            return carry

        lax.fori_loop(0, _CH, body, 0)
        pltpu.sync_copy(fe_v, out_hbm.at[:, pl.ds(off, _BLK)])


def _mlp_body(f_ref, w1_ref, b1_ref, w2_ref, b2_ref, o_ref):
    f = f_ref[...]
    h = jnp.dot(w1_ref[...], f, preferred_element_type=jnp.float32)
    h = jnp.maximum(h + b1_ref[...], 0.0)
    g = jnp.dot(w2_ref[...], h, preferred_element_type=jnp.float32)
    g = g + b2_ref[...]
    o_ref[...] = 1.0 / (1.0 + jnp.exp(-g))


def kernel(x, codebooks, dec_w, dec_b, w1, b1, w2, b2):
    x0 = x[:, 0]
    x1 = x[:, 1]
    tab = codebooks.reshape(_NUM_LODS * _TABLE)
    w1t = (w1 * dec_w[:, None]).T
    b1t = (b1 + dec_b @ w1).reshape(_HIDDEN, 1)
    w2t = w2.T
    b2t = b2.reshape(_OUT, 1)

    lat_t = _sc_latents(x0, x1, tab)

    bn = 32768
    out_t = pl.pallas_call(
        _mlp_body,
        grid=(_N // bn,),
        in_specs=[
            pl.BlockSpec((_NUM_LODS, bn), lambda i: (0, i)),
            pl.BlockSpec((_HIDDEN, _NUM_LODS), lambda i: (0, 0)),
            pl.BlockSpec((_HIDDEN, 1), lambda i: (0, 0)),
            pl.BlockSpec((_OUT, _HIDDEN), lambda i: (0, 0)),
            pl.BlockSpec((_OUT, 1), lambda i: (0, 0)),
        ],
        out_specs=pl.BlockSpec((_OUT, bn), lambda i: (0, i)),
        out_shape=jax.ShapeDtypeStruct((_OUT, _N), jnp.float32),
    )(lat_t, w1t, b1t, w2t, b2t)
    return out_t.T


# speedup vs baseline: 1181.1126x; 1.3140x over previous
import functools

import jax
import jax.numpy as jnp
import numpy as np
from jax import lax
from jax.experimental import pallas as pl
from jax.experimental.pallas import tpu as pltpu
from jax.experimental.pallas import tpu_sc as plsc

_NUM_LODS = 16
_TABLE = 4096
_N = 262144
_HIDDEN = 16
_OUT = 3
_MIN_RES, _MAX_RES = 16, 512

_bf = np.exp((np.log(_MAX_RES) - np.log(_MIN_RES)) / (_NUM_LODS - 1))
_RES = [int(np.floor(_MIN_RES * (_bf ** l))) for l in range(_NUM_LODS)]

_NC, _NS, _L = 2, 16, 16
_NW = _NC * _NS
_PPW = _N // _NW
_BLK = 2048
_CH = _BLK // _L

_mesh = plsc.VectorSubcoreMesh(core_axis_name="c", subcore_axis_name="s")


@functools.partial(
    pl.kernel,
    mesh=_mesh,
    compiler_params=pltpu.CompilerParams(needs_layout_passes=False),
    out_type=jax.ShapeDtypeStruct((_NUM_LODS, _N), jnp.float32),
    scratch_types=[
        pltpu.VMEM((_NUM_LODS * _TABLE,), jnp.float32),
        pltpu.VMEM((_BLK,), jnp.float32),
        pltpu.VMEM((_BLK,), jnp.float32),
        pltpu.VMEM((_NUM_LODS, _BLK), jnp.float32),
    ],
)
def _sc_latents(x0_hbm, x1_hbm, tab_hbm, out_hbm, tab_v, x0_v, x1_v, fe_v):
    wid = lax.axis_index("s") * _NC + lax.axis_index("c")
    pltpu.sync_copy(tab_hbm, tab_v)
    kmul = jnp.int32(2654435761 - (1 << 32))
    mask = jnp.int32(_TABLE - 1)

    for blk in range(_PPW // _BLK):
        off = wid * _PPW + blk * _BLK
        pltpu.sync_copy(x0_hbm.at[pl.ds(off, _BLK)], x0_v)
        pltpu.sync_copy(x1_hbm.at[pl.ds(off, _BLK)], x1_v)

        @plsc.parallel_loop(0, _CH, unroll=2)
        def body(i):
            xv = x0_v[pl.ds(i * _L, _L)]
            yv = x1_v[pl.ds(i * _L, _L)]
            for l in range(_NUM_LODS):
                r = float(_RES[l])
                px = xv * r
                py = yv * r
                ix = px.astype(jnp.int32)
                iy = py.astype(jnp.int32)
                fx = px - ix.astype(jnp.float32)
                fy = py - iy.astype(jnp.float32)
                hy0 = iy * kmul
                hy1 = hy0 + kmul
                ix1 = ix + jnp.int32(1)
                off_l = jnp.int32(l * _TABLE)
                i00 = ((ix ^ hy0) & mask) + off_l
                i10 = ((ix1 ^ hy0) & mask) + off_l
                i01 = ((ix ^ hy1) & mask) + off_l
                i11 = ((ix1 ^ hy1) & mask) + off_l
                f00 = plsc.load_gather(tab_v, [i00])
                f10 = plsc.load_gather(tab_v, [i10])
                f01 = plsc.load_gather(tab_v, [i01])
                f11 = plsc.load_gather(tab_v, [i11])
                a = f00 + fx * (f10 - f00)
                b = f01 + fx * (f11 - f01)
                fe_v[l, pl.ds(i * _L, _L)] = a + fy * (b - a)

        pltpu.sync_copy(fe_v, out_hbm.at[:, pl.ds(off, _BLK)])


def _mlp_body(f_ref, w1_ref, b1_ref, w2_ref, b2_ref, o_ref):
    f = f_ref[...]
    h = jnp.dot(w1_ref[...], f, preferred_element_type=jnp.float32)
    h = jnp.maximum(h + b1_ref[...], 0.0)
    g = jnp.dot(w2_ref[...], h, preferred_element_type=jnp.float32)
    g = g + b2_ref[...]
    o_ref[...] = 1.0 / (1.0 + jnp.exp(-g))


def kernel(x, codebooks, dec_w, dec_b, w1, b1, w2, b2):
    x0 = x[:, 0]
    x1 = x[:, 1]
    tab = codebooks.reshape(_NUM_LODS * _TABLE)
    w1t = (w1 * dec_w[:, None]).T
    b1t = (b1 + dec_b @ w1).reshape(_HIDDEN, 1)
    w2t = w2.T
    b2t = b2.reshape(_OUT, 1)

    lat_t = _sc_latents(x0, x1, tab)

    bn = 32768
    out_t = pl.pallas_call(
        _mlp_body,
        grid=(_N // bn,),
        in_specs=[
            pl.BlockSpec((_NUM_LODS, bn), lambda i: (0, i)),
            pl.BlockSpec((_HIDDEN, _NUM_LODS), lambda i: (0, 0)),
            pl.BlockSpec((_HIDDEN, 1), lambda i: (0, 0)),
            pl.BlockSpec((_OUT, _HIDDEN), lambda i: (0, 0)),
            pl.BlockSpec((_OUT, 1), lambda i: (0, 0)),
        ],
        out_specs=pl.BlockSpec((_OUT, bn), lambda i: (0, i)),
        out_shape=jax.ShapeDtypeStruct((_OUT, _N), jnp.float32),
    )(lat_t, w1t, b1t, w2t, b2t)
    return out_t.T


# speedup vs baseline: 1219.1104x; 1.0322x over previous
import functools

import jax
import jax.numpy as jnp
import numpy as np
from jax import lax
from jax.experimental import pallas as pl
from jax.experimental.pallas import tpu as pltpu
from jax.experimental.pallas import tpu_sc as plsc

_NUM_LODS = 16
_TABLE = 4096
_N = 262144
_HIDDEN = 16
_OUT = 3
_MIN_RES, _MAX_RES = 16, 512

_bf = np.exp((np.log(_MAX_RES) - np.log(_MIN_RES)) / (_NUM_LODS - 1))
_RES = [int(np.floor(_MIN_RES * (_bf ** l))) for l in range(_NUM_LODS)]

_NC, _NS, _L = 2, 16, 16
_NW = _NC * _NS
_PPW = _N // _NW
_BLK = 2048
_CH = _BLK // _L

_mesh = plsc.VectorSubcoreMesh(core_axis_name="c", subcore_axis_name="s")


@functools.partial(
    pl.kernel,
    mesh=_mesh,
    compiler_params=pltpu.CompilerParams(needs_layout_passes=False),
    out_type=jax.ShapeDtypeStruct((_NUM_LODS, _N), jnp.float32),
    scratch_types=[
        pltpu.VMEM((_NUM_LODS * _TABLE,), jnp.float32),
        pltpu.VMEM((_BLK,), jnp.float32),
        pltpu.VMEM((_BLK,), jnp.float32),
        pltpu.VMEM((_NUM_LODS, _BLK), jnp.float32),
    ],
)
def _sc_latents(x0_hbm, x1_hbm, tab_hbm, out_hbm, tab_v, x0_v, x1_v, fe_v):
    wid = lax.axis_index("s") * _NC + lax.axis_index("c")
    pltpu.sync_copy(tab_hbm, tab_v)
    kmul = jnp.int32(2654435761 - (1 << 32))
    mask = jnp.int32(_TABLE - 1)

    for blk in range(_PPW // _BLK):
        off = wid * _PPW + blk * _BLK
        pltpu.sync_copy(x0_hbm.at[pl.ds(off, _BLK)], x0_v)
        pltpu.sync_copy(x1_hbm.at[pl.ds(off, _BLK)], x1_v)

        @plsc.parallel_loop(0, _CH, unroll=4)
        def body(i):
            xv = x0_v[pl.ds(i * _L, _L)]
            yv = x1_v[pl.ds(i * _L, _L)]
            for l in range(_NUM_LODS):
                r = float(_RES[l])
                px = xv * r
                py = yv * r
                ix = px.astype(jnp.int32)
                iy = py.astype(jnp.int32)
                fx = px - ix.astype(jnp.float32)
                fy = py - iy.astype(jnp.float32)
                hy0 = iy * kmul
                hy1 = hy0 + kmul
                ix1 = ix + jnp.int32(1)
                off_l = jnp.int32(l * _TABLE)
                i00 = ((ix ^ hy0) & mask) + off_l
                i10 = ((ix1 ^ hy0) & mask) + off_l
                i01 = ((ix ^ hy1) & mask) + off_l
                i11 = ((ix1 ^ hy1) & mask) + off_l
                f00 = plsc.load_gather(tab_v, [i00])
                f10 = plsc.load_gather(tab_v, [i10])
                f01 = plsc.load_gather(tab_v, [i01])
                f11 = plsc.load_gather(tab_v, [i11])
                a = f00 + fx * (f10 - f00)
                b = f01 + fx * (f11 - f01)
                fe_v[l, pl.ds(i * _L, _L)] = a + fy * (b - a)

        pltpu.sync_copy(fe_v, out_hbm.at[:, pl.ds(off, _BLK)])


def _mlp_body(f_ref, w1_ref, b1_ref, w2_ref, b2_ref, o_ref):
    f = f_ref[...]
    h = jnp.dot(w1_ref[...], f, preferred_element_type=jnp.float32)
    h = jnp.maximum(h + b1_ref[...], 0.0)
    g = jnp.dot(w2_ref[...], h, preferred_element_type=jnp.float32)
    g = g + b2_ref[...]
    o_ref[...] = 1.0 / (1.0 + jnp.exp(-g))


def kernel(x, codebooks, dec_w, dec_b, w1, b1, w2, b2):
    x0 = x[:, 0]
    x1 = x[:, 1]
    tab = codebooks.reshape(_NUM_LODS * _TABLE)
    w1t = (w1 * dec_w[:, None]).T
    b1t = (b1 + dec_b @ w1).reshape(_HIDDEN, 1)
    w2t = w2.T
    b2t = b2.reshape(_OUT, 1)

    lat_t = _sc_latents(x0, x1, tab)

    bn = 32768
    out_t = pl.pallas_call(
        _mlp_body,
        grid=(_N // bn,),
        in_specs=[
            pl.BlockSpec((_NUM_LODS, bn), lambda i: (0, i)),
            pl.BlockSpec((_HIDDEN, _NUM_LODS), lambda i: (0, 0)),
            pl.BlockSpec((_HIDDEN, 1), lambda i: (0, 0)),
            pl.BlockSpec((_OUT, _HIDDEN), lambda i: (0, 0)),
            pl.BlockSpec((_OUT, 1), lambda i: (0, 0)),
        ],
        out_specs=pl.BlockSpec((_OUT, bn), lambda i: (0, i)),
        out_shape=jax.ShapeDtypeStruct((_OUT, _N), jnp.float32),
    )(lat_t, w1t, b1t, w2t, b2t)
    return out_t.T


# speedup vs baseline: 1267.4945x; 1.0397x over previous
import functools

import jax
import jax.numpy as jnp
import numpy as np
from jax import lax
from jax.experimental import pallas as pl
from jax.experimental.pallas import tpu as pltpu
from jax.experimental.pallas import tpu_sc as plsc

_NUM_LODS = 16
_TABLE = 4096
_N = 262144
_HIDDEN = 16
_OUT = 3
_MIN_RES, _MAX_RES = 16, 512

_bf = np.exp((np.log(_MAX_RES) - np.log(_MIN_RES)) / (_NUM_LODS - 1))
_RES = [int(np.floor(_MIN_RES * (_bf ** l))) for l in range(_NUM_LODS)]

_NC, _NS, _L = 2, 16, 16
_NW = _NC * _NS
_PPW = _N // _NW
_BLK = 2048
_CH = _BLK // _L

_mesh = plsc.VectorSubcoreMesh(core_axis_name="c", subcore_axis_name="s")


@functools.partial(
    pl.kernel,
    mesh=_mesh,
    compiler_params=pltpu.CompilerParams(needs_layout_passes=False),
    out_type=jax.ShapeDtypeStruct((_NUM_LODS, _N), jnp.float32),
    scratch_types=[
        pltpu.VMEM((_NUM_LODS * _TABLE,), jnp.float32),
        pltpu.VMEM((_BLK,), jnp.float32),
        pltpu.VMEM((_BLK,), jnp.float32),
        pltpu.VMEM((_NUM_LODS, _BLK), jnp.float32),
    ],
)
def _sc_latents(x0_hbm, x1_hbm, tab_hbm, out_hbm, tab_v, x0_v, x1_v, fe_v):
    wid = lax.axis_index("s") * _NC + lax.axis_index("c")
    pltpu.sync_copy(tab_hbm, tab_v)
    kmul = jnp.int32(2654435761 - (1 << 32))
    mask = jnp.int32(_TABLE - 1)

    def outer(blk, carry):
        off = wid * _PPW + blk * _BLK
        pltpu.sync_copy(x0_hbm.at[pl.ds(off, _BLK)], x0_v)
        pltpu.sync_copy(x1_hbm.at[pl.ds(off, _BLK)], x1_v)

        @plsc.parallel_loop(0, _CH, unroll=4)
        def body(i):
            xv = x0_v[pl.ds(i * _L, _L)]
            yv = x1_v[pl.ds(i * _L, _L)]
            for l in range(_NUM_LODS):
                r = float(_RES[l])
                px = xv * r
                py = yv * r
                ix = px.astype(jnp.int32)
                iy = py.astype(jnp.int32)
                fx = px - ix.astype(jnp.float32)
                fy = py - iy.astype(jnp.float32)
                hy0 = iy * kmul
                hy1 = hy0 + kmul
                ix1 = ix + jnp.int32(1)
                off_l = jnp.int32(l * _TABLE)
                i00 = ((ix ^ hy0) & mask) + off_l
                i10 = ((ix1 ^ hy0) & mask) + off_l
                i01 = ((ix ^ hy1) & mask) + off_l
                i11 = ((ix1 ^ hy1) & mask) + off_l
                f00 = plsc.load_gather(tab_v, [i00])
                f10 = plsc.load_gather(tab_v, [i10])
                f01 = plsc.load_gather(tab_v, [i01])
                f11 = plsc.load_gather(tab_v, [i11])
                a = f00 + fx * (f10 - f00)
                b = f01 + fx * (f11 - f01)
                fe_v[l, pl.ds(i * _L, _L)] = a + fy * (b - a)

        pltpu.sync_copy(fe_v, out_hbm.at[:, pl.ds(off, _BLK)])
        return carry

    lax.fori_loop(0, _PPW // _BLK, outer, 0)


def _mlp_body(f_ref, w1_ref, b1_ref, w2_ref, b2_ref, o_ref):
    f = f_ref[...]
    h = jnp.dot(w1_ref[...], f, preferred_element_type=jnp.float32)
    h = jnp.maximum(h + b1_ref[...], 0.0)
    g = jnp.dot(w2_ref[...], h, preferred_element_type=jnp.float32)
    g = g + b2_ref[...]
    o_ref[...] = 1.0 / (1.0 + jnp.exp(-g))


def kernel(x, codebooks, dec_w, dec_b, w1, b1, w2, b2):
    x0 = x[:, 0]
    x1 = x[:, 1]
    tab = codebooks.reshape(_NUM_LODS * _TABLE)
    w1t = (w1 * dec_w[:, None]).T
    b1t = (b1 + dec_b @ w1).reshape(_HIDDEN, 1)
    w2t = w2.T
    b2t = b2.reshape(_OUT, 1)

    lat_t = _sc_latents(x0, x1, tab)

    bn = 32768
    out_t = pl.pallas_call(
        _mlp_body,
        grid=(_N // bn,),
        in_specs=[
            pl.BlockSpec((_NUM_LODS, bn), lambda i: (0, i)),
            pl.BlockSpec((_HIDDEN, _NUM_LODS), lambda i: (0, 0)),
            pl.BlockSpec((_HIDDEN, 1), lambda i: (0, 0)),
            pl.BlockSpec((_OUT, _HIDDEN), lambda i: (0, 0)),
            pl.BlockSpec((_OUT, 1), lambda i: (0, 0)),
        ],
        out_specs=pl.BlockSpec((_OUT, bn), lambda i: (0, i)),
        out_shape=jax.ShapeDtypeStruct((_OUT, _N), jnp.float32),
    )(lat_t, w1t, b1t, w2t, b2t)
    return out_t.T
